# baseline (speedup 1.0000x reference)
import numpy as np
import jax
import jax.numpy as jnp
from jax import lax
from jax.experimental import pallas as pl
from jax.experimental.pallas import tpu as pltpu

B = 128
V = 100000
VPAD = 102400
R = 16
C = 6400
NBLK = B // R
NCHUNK = VPAD // C
LANES = 128

TINY = np.float32(np.finfo(np.float32).tiny)
NEG_INF = np.float32(-np.inf)
INT_MAX = np.int32(np.iinfo(np.int32).max)

_KS0 = np.int32(0)
_KS1 = np.int32(42)
_KS2 = np.int32(np.uint32(0) ^ np.uint32(42) ^ np.uint32(0x1BD11BDA))
_ROT = ((13, 15, 26, 6), (17, 29, 16, 24))


def _rotl(x, r):
    return lax.shift_left(x, np.int32(r)) | lax.shift_right_logical(
        x, np.int32(32 - r))


def _threefry_bits(ctr):
    ks = (_KS0, _KS1, _KS2)
    x1 = ctr + ks[1]
    x0 = jnp.zeros_like(ctr) + ks[0]
    for i in range(5):
        for r in _ROT[i % 2]:
            x0 = x0 + x1
            x1 = _rotl(x1, r)
            x1 = x0 ^ x1
        x0 = x0 + ks[(i + 1) % 3]
        x1 = x1 + ks[(i + 2) % 3] + np.int32(i + 1)
    return x0 ^ x1


def _body(logits_ref, out_ref, bv_ref, bi_ref):
    rblk = pl.program_id(0)
    cblk = pl.program_id(1)

    @pl.when(cblk == 0)
    def _init():
        bv_ref[...] = jnp.full((R, LANES), NEG_INF, jnp.float32)
        bi_ref[...] = jnp.zeros((R, LANES), jnp.int32)

    row0 = rblk * R
    col0 = cblk * C
    rows = lax.broadcasted_iota(jnp.int32, (R, C), 0) + row0
    cols = lax.broadcasted_iota(jnp.int32, (R, C), 1) + col0
    bits = _threefry_bits(rows * np.int32(V) + cols)

    fb = lax.shift_right_logical(bits, np.int32(9)) | np.int32(0x3F800000)
    f = lax.bitcast_convert_type(fb, jnp.float32) - np.float32(1.0)
    u = jnp.maximum(f + TINY, TINY)
    g = -jnp.log(-jnp.log(u))
    vals = g + logits_ref[...]

    bv = bv_ref[...]
    bi = bi_ref[...]
    lane = lax.broadcasted_iota(jnp.int32, (R, LANES), 1)
    for j in range(C // LANES):
        slab = vals[:, j * LANES:(j + 1) * LANES]
        sidx = lane + (col0 + np.int32(j * LANES))
        better = slab > bv
        bi = jnp.where(better, sidx, bi)
        bv = jnp.where(better, slab, bv)
    bv_ref[...] = bv
    bi_ref[...] = bi

    @pl.when(cblk == NCHUNK - 1)
    def _finalize():
        fv = bv_ref[...]
        fi = bi_ref[...]
        m = jnp.max(fv, axis=1, keepdims=True)
        cand = jnp.where(fv == m, fi, INT_MAX)
        out_ref[0, :, :] = jnp.min(cand, axis=1, keepdims=True)


@jax.jit
def kernel(logits):
    pad = jnp.pad(logits, ((0, 0), (0, VPAD - V)), constant_values=NEG_INF)
    out = pl.pallas_call(
        _body,
        grid=(NBLK, NCHUNK),
        in_specs=[pl.BlockSpec((R, C), lambda r, c: (r, c))],
        out_specs=pl.BlockSpec((1, R, 1), lambda r, c: (r, 0, 0)),
        out_shape=jax.ShapeDtypeStruct((NBLK, R, 1), jnp.int32),
        scratch_shapes=[
            pltpu.VMEM((R, LANES), jnp.float32),
            pltpu.VMEM((R, LANES), jnp.int32),
        ],
        compiler_params=pltpu.CompilerParams(
            dimension_semantics=("arbitrary", "arbitrary")),
    )(pad)
    return out.reshape(B).astype(jnp.int64)


# speedup vs baseline: 1.2919x; 1.2919x over previous
import numpy as np
import jax
import jax.numpy as jnp
from jax import lax
from jax.experimental import pallas as pl
from jax.experimental.pallas import tpu as pltpu

B = 128
V = 100000
VPAD = 102400
R = 16
C = 6400
ST = 640
NBLK = B // R
NCHUNK = VPAD // C
NST = C // ST
LANES = 128

TINY = np.float32(np.finfo(np.float32).tiny)
NEG_INF = np.float32(-np.inf)
INT_MAX = np.int32(np.iinfo(np.int32).max)

_K1 = np.int32(42)
_K2 = np.int32(np.uint32(0) ^ np.uint32(42) ^ np.uint32(0x1BD11BDA))
_ROT = ((13, 15, 26, 6), (17, 29, 16, 24))


def _rotl(x, r):
    return lax.shift_left(x, np.int32(r)) | lax.shift_right_logical(
        x, np.int32(32 - r))


def _threefry_bits(x1):
    x0 = x1
    x1 = _rotl(x1, 13) ^ x0
    for r in (15, 26, 6):
        x0 = x0 + x1
        x1 = _rotl(x1, r) ^ x0
    x0 = x0 + _K1
    x1 = x1 + np.int32(_K2 + 1)
    for r in (17, 29, 16, 24):
        x0 = x0 + x1
        x1 = _rotl(x1, r) ^ x0
    x0 = x0 + _K2
    x1 = x1 + np.int32(2)
    for r in (13, 15, 26, 6):
        x0 = x0 + x1
        x1 = _rotl(x1, r) ^ x0
    x1 = x1 + np.int32(_K1 + 3)
    for r in (17, 29, 16, 24):
        x0 = x0 + x1
        x1 = _rotl(x1, r) ^ x0
    x0 = x0 + _K1
    x1 = x1 + np.int32(_K2 + 4)
    for r in (13, 15, 26, 6):
        x0 = x0 + x1
        x1 = _rotl(x1, r) ^ x0
    x0 = x0 + _K2
    x1 = x1 + np.int32(5)
    return x0 ^ x1


def _body(logits_ref, out_ref, bv_ref, bi_ref):
    rblk = pl.program_id(0)
    cblk = pl.program_id(1)

    @pl.when(cblk == 0)
    def _init():
        bv_ref[...] = jnp.full((R, LANES), NEG_INF, jnp.float32)
        bi_ref[...] = jnp.zeros((R, LANES), jnp.int32)

    row0 = rblk * R
    col0 = cblk * C
    ctr_base = (lax.broadcasted_iota(jnp.int32, (R, ST), 0) + row0) * np.int32(
        V) + lax.broadcasted_iota(jnp.int32, (R, ST), 1)

    bv = bv_ref[...]
    bi = bi_ref[...]
    lane = lax.broadcasted_iota(jnp.int32, (R, LANES), 1)
    for st in range(NST):
        off = col0 + np.int32(st * ST)
        bits = _threefry_bits(ctr_base + (off + np.int32(42)))
        fb = lax.shift_right_logical(bits, np.int32(9)) | np.int32(0x3F800000)
        f = lax.bitcast_convert_type(fb, jnp.float32) - np.float32(1.0)
        u = jnp.maximum(f, TINY)
        e = -jnp.log(u)
        vals = logits_ref[:, st * ST:(st + 1) * ST] - jnp.log(e)
        for j in range(ST // LANES):
            slab = vals[:, j * LANES:(j + 1) * LANES]
            sidx = lane + (off + np.int32(j * LANES))
            better = slab > bv
            bi = jnp.where(better, sidx, bi)
            bv = jnp.where(better, slab, bv)
    bv_ref[...] = bv
    bi_ref[...] = bi

    @pl.when(cblk == NCHUNK - 1)
    def _finalize():
        fv = bv_ref[...]
        fi = bi_ref[...]
        m = jnp.max(fv, axis=1, keepdims=True)
        cand = jnp.where(fv == m, fi, INT_MAX)
        out_ref[0, :, :] = jnp.min(cand, axis=1, keepdims=True)


@jax.jit
def kernel(logits):
    pad = jnp.pad(logits, ((0, 0), (0, VPAD - V)), constant_values=NEG_INF)
    out = pl.pallas_call(
        _body,
        grid=(NBLK, NCHUNK),
        in_specs=[pl.BlockSpec((R, C), lambda r, c: (r, c))],
        out_specs=pl.BlockSpec((1, R, 1), lambda r, c: (r, 0, 0)),
        out_shape=jax.ShapeDtypeStruct((NBLK, R, 1), jnp.int32),
        scratch_shapes=[
            pltpu.VMEM((R, LANES), jnp.float32),
            pltpu.VMEM((R, LANES), jnp.int32),
        ],
        compiler_params=pltpu.CompilerParams(
            dimension_semantics=("arbitrary", "arbitrary")),
    )(pad)
    return out.reshape(B).astype(jnp.int64)


# speedup vs baseline: 1.4986x; 1.1600x over previous
import numpy as np
import jax
import jax.numpy as jnp
from jax import lax
from jax.experimental import pallas as pl
from jax.experimental.pallas import tpu as pltpu

B = 128
V = 100000
R = 16
C = 12800
ST = 640
NBLK = B // R
NCHUNK = 8
NST = C // ST
LANES = 128

TINY = np.float32(np.finfo(np.float32).tiny)
NEG_INF = np.float32(-np.inf)
INT_MAX = np.int32(np.iinfo(np.int32).max)

_K1 = np.int32(42)
_K2 = np.int32(np.uint32(0) ^ np.uint32(42) ^ np.uint32(0x1BD11BDA))


def _rotl(x, r):
    return lax.shift_left(x, np.int32(r)) | lax.shift_right_logical(
        x, np.int32(32 - r))


def _threefry_bits(x1):
    x0 = x1
    x1 = _rotl(x1, 13) ^ x0
    for r in (15, 26, 6):
        x0 = x0 + x1
        x1 = _rotl(x1, r) ^ x0
    x0 = x0 + _K1
    x1 = x1 + np.int32(_K2 + 1)
    for r in (17, 29, 16, 24):
        x0 = x0 + x1
        x1 = _rotl(x1, r) ^ x0
    x0 = x0 + _K2
    x1 = x1 + np.int32(2)
    for r in (13, 15, 26, 6):
        x0 = x0 + x1
        x1 = _rotl(x1, r) ^ x0
    x1 = x1 + np.int32(_K1 + 3)
    for r in (17, 29, 16, 24):
        x0 = x0 + x1
        x1 = _rotl(x1, r) ^ x0
    x0 = x0 + _K1
    x1 = x1 + np.int32(_K2 + 4)
    for r in (13, 15, 26, 6):
        x0 = x0 + x1
        x1 = _rotl(x1, r) ^ x0
    x0 = x0 + _K2
    x1 = x1 + np.int32(5)
    return x0 ^ x1


def _scan_chunk(logits_ref, bv, bi, ctr_base, lane, col0, masked):
    for st in range(NST):
        off = col0 + np.int32(st * ST)
        bits = _threefry_bits(ctr_base + (off + np.int32(42)))
        fb = lax.shift_right_logical(bits, np.int32(9)) | np.int32(0x3F800000)
        f = lax.bitcast_convert_type(fb, jnp.float32) - np.float32(1.0)
        u = jnp.maximum(f, TINY)
        e = -jnp.log(u)
        vals = logits_ref[:, st * ST:(st + 1) * ST] - jnp.log(e)
        for j in range(ST // LANES):
            slab = vals[:, j * LANES:(j + 1) * LANES]
            sidx = lane + (off + np.int32(j * LANES))
            better = slab > bv
            if masked:
                better = better & (sidx < V)
            bi = jnp.where(better, sidx, bi)
            bv = jnp.where(better, slab, bv)
    return bv, bi


def _body(logits_ref, out_ref, bv_ref, bi_ref):
    rblk = pl.program_id(0)
    cblk = pl.program_id(1)

    @pl.when(cblk == 0)
    def _init():
        bv_ref[...] = jnp.full((R, LANES), NEG_INF, jnp.float32)
        bi_ref[...] = jnp.zeros((R, LANES), jnp.int32)

    row0 = rblk * R
    col0 = cblk * C
    ctr_base = (lax.broadcasted_iota(jnp.int32, (R, ST), 0) + row0) * np.int32(
        V) + lax.broadcasted_iota(jnp.int32, (R, ST), 1)
    lane = lax.broadcasted_iota(jnp.int32, (R, LANES), 1)

    @pl.when(cblk < NCHUNK - 1)
    def _main():
        bv, bi = _scan_chunk(logits_ref, bv_ref[...], bi_ref[...], ctr_base,
                             lane, col0, masked=False)
        bv_ref[...] = bv
        bi_ref[...] = bi

    @pl.when(cblk == NCHUNK - 1)
    def _last():
        bv, bi = _scan_chunk(logits_ref, bv_ref[...], bi_ref[...], ctr_base,
                             lane, col0, masked=True)
        m = jnp.max(bv, axis=1, keepdims=True)
        cand = jnp.where(bv == m, bi, INT_MAX)
        out_ref[0, :, :] = jnp.min(cand, axis=1, keepdims=True)


@jax.jit
def kernel(logits):
    out = pl.pallas_call(
        _body,
        grid=(NBLK, NCHUNK),
        in_specs=[pl.BlockSpec((R, C), lambda r, c: (r, c))],
        out_specs=pl.BlockSpec((1, R, 1), lambda r, c: (r, 0, 0)),
        out_shape=jax.ShapeDtypeStruct((NBLK, R, 1), jnp.int32),
        scratch_shapes=[
            pltpu.VMEM((R, LANES), jnp.float32),
            pltpu.VMEM((R, LANES), jnp.int32),
        ],
        compiler_params=pltpu.CompilerParams(
            dimension_semantics=("arbitrary", "arbitrary")),
    )(logits)
    return out.reshape(B).astype(jnp.int64)


# speedup vs baseline: 1.5502x; 1.0344x over previous
import numpy as np
import jax
import jax.numpy as jnp
from jax import lax
from jax.experimental import pallas as pl
from jax.experimental.pallas import tpu as pltpu

B = 128
V = 100000
R = 16
ST = 640
NST = 157
CPAD = NST * ST
NBLK = B // R
LANES = 128

TINY = np.float32(np.finfo(np.float32).tiny)
NEG_INF = np.float32(-np.inf)
INT_MAX = np.int32(np.iinfo(np.int32).max)

_K1 = np.int32(42)
_K2 = np.int32(np.uint32(0) ^ np.uint32(42) ^ np.uint32(0x1BD11BDA))


def _rotl(x, r):
    return lax.shift_left(x, np.int32(r)) | lax.shift_right_logical(
        x, np.int32(32 - r))


def _threefry_bits(x1):
    x0 = x1
    x1 = _rotl(x1, 13) ^ x0
    for r in (15, 26, 6):
        x0 = x0 + x1
        x1 = _rotl(x1, r) ^ x0
    x0 = x0 + _K1
    x1 = x1 + np.int32(_K2 + 1)
    for r in (17, 29, 16, 24):
        x0 = x0 + x1
        x1 = _rotl(x1, r) ^ x0
    x0 = x0 + _K2
    x1 = x1 + np.int32(2)
    for r in (13, 15, 26, 6):
        x0 = x0 + x1
        x1 = _rotl(x1, r) ^ x0
    x1 = x1 + np.int32(_K1 + 3)
    for r in (17, 29, 16, 24):
        x0 = x0 + x1
        x1 = _rotl(x1, r) ^ x0
    x0 = x0 + _K1
    x1 = x1 + np.int32(_K2 + 4)
    for r in (13, 15, 26, 6):
        x0 = x0 + x1
        x1 = _rotl(x1, r) ^ x0
    x0 = x0 + _K2
    x1 = x1 + np.int32(5)
    return x0 ^ x1


def _body(logits_ref, out_ref):
    rblk = pl.program_id(0)
    row0 = rblk * R
    ctr_base = (lax.broadcasted_iota(jnp.int32, (R, ST), 0) + row0) * np.int32(
        V) + lax.broadcasted_iota(jnp.int32, (R, ST), 1)
    lane = lax.broadcasted_iota(jnp.int32, (R, LANES), 1)

    bv = jnp.full((R, LANES), NEG_INF, jnp.float32)
    bi = jnp.zeros((R, LANES), jnp.int32)
    for st in range(NST):
        off = np.int32(st * ST)
        bits = _threefry_bits(ctr_base + np.int32(off + 42))
        fb = lax.shift_right_logical(bits, np.int32(9)) | np.int32(0x3F800000)
        f = lax.bitcast_convert_type(fb, jnp.float32) - np.float32(1.0)
        u = jnp.maximum(f, TINY)
        e = -jnp.log(u)
        vals = logits_ref[:, st * ST:(st + 1) * ST] - jnp.log(e)
        masked = (st + 1) * ST > V
        for j in range(ST // LANES):
            slab = vals[:, j * LANES:(j + 1) * LANES]
            sidx = lane + np.int32(off + j * LANES)
            better = slab > bv
            if masked:
                better = better & (sidx < V)
            bi = jnp.where(better, sidx, bi)
            bv = jnp.where(better, slab, bv)

    m = jnp.max(bv, axis=1, keepdims=True)
    cand = jnp.where(bv == m, bi, INT_MAX)
    out_ref[0, :, :] = jnp.min(cand, axis=1, keepdims=True)


@jax.jit
def kernel(logits):
    out = pl.pallas_call(
        _body,
        grid=(NBLK,),
        in_specs=[pl.BlockSpec((R, CPAD), lambda r: (r, 0))],
        out_specs=pl.BlockSpec((1, R, 1), lambda r: (r, 0, 0)),
        out_shape=jax.ShapeDtypeStruct((NBLK, R, 1), jnp.int32),
        compiler_params=pltpu.CompilerParams(
            dimension_semantics=("arbitrary",)),
    )(logits)
    return out.reshape(B).astype(jnp.int64)


# speedup vs baseline: 1.5507x; 1.0003x over previous
import numpy as np
import jax
import jax.numpy as jnp
from jax import lax
from jax.experimental import pallas as pl
from jax.experimental.pallas import tpu as pltpu

B = 128
V = 100000
R = 16
ST = 640
NST = 156
TAIL = 256
CPAD = NST * ST + TAIL
NBLK = B // R
LANES = 128

TINY = np.float32(np.finfo(np.float32).tiny)
NEG_INF = np.float32(-np.inf)
INT_MAX = np.int32(np.iinfo(np.int32).max)

_K1 = np.int32(42)
_K2 = np.int32(np.uint32(0) ^ np.uint32(42) ^ np.uint32(0x1BD11BDA))


def _rotl(x, r):
    return lax.shift_left(x, np.int32(r)) | lax.shift_right_logical(
        x, np.int32(32 - r))


def _threefry_bits(x1):
    x0 = x1
    x1 = _rotl(x1, 13) ^ x0
    for r in (15, 26, 6):
        x0 = x0 + x1
        x1 = _rotl(x1, r) ^ x0
    x0 = x0 + _K1
    x1 = x1 + np.int32(_K2 + 1)
    for r in (17, 29, 16, 24):
        x0 = x0 + x1
        x1 = _rotl(x1, r) ^ x0
    x0 = x0 + _K2
    x1 = x1 + np.int32(2)
    for r in (13, 15, 26, 6):
        x0 = x0 + x1
        x1 = _rotl(x1, r) ^ x0
    x1 = x1 + np.int32(_K1 + 3)
    for r in (17, 29, 16, 24):
        x0 = x0 + x1
        x1 = _rotl(x1, r) ^ x0
    x0 = x0 + _K1
    x1 = x1 + np.int32(_K2 + 4)
    for r in (13, 15, 26, 6):
        x0 = x0 + x1
        x1 = _rotl(x1, r) ^ x0
    x0 = x0 + _K2
    x1 = x1 + np.int32(5)
    return x0 ^ x1


def _body(logits_ref, out_ref):
    rblk = pl.program_id(0)
    row0 = rblk * R
    ctr_base = (lax.broadcasted_iota(jnp.int32, (R, ST), 0) + row0) * np.int32(
        V) + lax.broadcasted_iota(jnp.int32, (R, ST), 1)
    lane = lax.broadcasted_iota(jnp.int32, (R, LANES), 1)

    bv = jnp.full((R, LANES), NEG_INF, jnp.float32)
    bi = jnp.zeros((R, LANES), jnp.int32)
    for st in range(NST + 1):
        width = ST if st < NST else TAIL
        off = np.int32(st * ST)
        base = ctr_base if st < NST else ctr_base[:, :TAIL]
        bits = _threefry_bits(base + np.int32(off + 42))
        fb = lax.shift_right_logical(bits, np.int32(9)) | np.int32(0x3F800000)
        f = lax.bitcast_convert_type(fb, jnp.float32) - np.float32(1.0)
        u = jnp.maximum(f, TINY)
        e = -jnp.log(u)
        vals = logits_ref[:, st * ST:st * ST + width] - jnp.log(e)
        for j in range(width // LANES):
            slab = vals[:, j * LANES:(j + 1) * LANES]
            sidx = lane + np.int32(off + j * LANES)
            better = slab > bv
            if st * ST + (j + 1) * LANES > V:
                better = better & (sidx < V)
            bi = jnp.where(better, sidx, bi)
            bv = jnp.where(better, slab, bv)

    m = jnp.max(bv, axis=1, keepdims=True)
    cand = jnp.where(bv == m, bi, INT_MAX)
    out_ref[0, :, :] = jnp.min(cand, axis=1, keepdims=True)


@jax.jit
def kernel(logits):
    out = pl.pallas_call(
        _body,
        grid=(NBLK,),
        in_specs=[pl.BlockSpec((R, CPAD), lambda r: (r, 0))],
        out_specs=pl.BlockSpec((1, R, 1), lambda r: (r, 0, 0)),
        out_shape=jax.ShapeDtypeStruct((NBLK, R, 1), jnp.int32),
        compiler_params=pltpu.CompilerParams(
            dimension_semantics=("arbitrary",)),
    )(logits)
    return out.reshape(B).astype(jnp.int64)
